# Initial kernel scaffold; baseline (speedup 1.0000x reference)
import jax
import jax.numpy as jnp
from jax.experimental import pallas as pl


def kernel(clouds):
    raise NotImplementedError("write your pallas kernel here")


# speedup vs baseline: 14.4269x; 14.4269x over previous
import jax
import jax.numpy as jnp
from jax import lax
from jax.experimental import pallas as pl
from jax.experimental.pallas import tpu as pltpu
from jax.experimental.pallas import tpu_sc as plsc

NB = 65536
TBL = 65664
RZ = TBL // 16
NPAD = 122880
CH = NPAD // 32
NSTREAM = CH // 128
L = 16
NC, NS = 2, 16
RB = NB // (NC * NS)

_mesh = plsc.VectorSubcoreMesh(core_axis_name="c", subcore_axis_name="s")


def _scatter_kernel(chans, zeros):
  @pl.kernel(
      out_type=jax.ShapeDtypeStruct((NC, TBL, 8), jnp.float32),
      mesh=_mesh,
      compiler_params=pltpu.CompilerParams(needs_layout_passes=False, use_tc_tiling_on_sc=False),
      scratch_types=[
          pltpu.VMEM_SHARED((TBL, 8), jnp.float32),
          pltpu.VMEM((CH, 8), jnp.float32),
          pltpu.VMEM((CH,), jnp.float32),
          pltpu.VMEM((CH,), jnp.float32),
          pltpu.VMEM((CH,), jnp.float32),
          pltpu.VMEM((CH,), jnp.float32),
          pltpu.VMEM((NSTREAM, 128), jnp.int32),
      ],
  )
  def body(chans_hbm, zeros_hbm, partials, table, feat, cx, cy, cz, cw, idx):
    c = lax.axis_index("c")
    s = lax.axis_index("s")
    wid = s * NC + c
    z0 = pl.multiple_of(s * RZ, 8)
    pltpu.sync_copy(zeros_hbm, table.at[pl.ds(z0, RZ)])
    plsc.subcore_barrier()

    iota = lax.iota(jnp.int32, L)
    cols = [jnp.full((L,), k, jnp.int32) for k in range(5)]
    off = pl.multiple_of(wid * CH, CH)
    for b in range(4):
      for k, cb in enumerate((cx, cy, cz, cw)):
        pltpu.sync_copy(chans_hbm.at[4 * b + k, pl.ds(off, CH)], cb)

      def step(i, carry):
        x = cx[pl.ds(i * L, L)]
        y = cy[pl.ds(i * L, L)]
        z = cz[pl.ds(i * L, L)]
        w = cw[pl.ds(i * L, L)]
        fx = (x + jnp.float32(51.2)) / jnp.float32(0.4)
        fy = (y + jnp.float32(51.2)) / jnp.float32(0.4)
        fz = (z + jnp.float32(5.0)) / jnp.float32(8.0)
        valid = ((fx >= 0.0) & (fx < 256.0)
                 & (fy >= 0.0) & (fy < 256.0)
                 & (fz >= 0.0) & (fz < 1.0))
        bx = fx.astype(jnp.int32)
        by = fy.astype(jnp.int32)
        bin_ = jnp.where(valid, by * 256 + bx, NB)
        row = jnp.full((L,), i * L, jnp.int32) + iota
        plsc.store_scatter(feat, [row, cols[0]], jnp.where(valid, x, 0.0))
        plsc.store_scatter(feat, [row, cols[1]], jnp.where(valid, y, 0.0))
        plsc.store_scatter(feat, [row, cols[2]], jnp.where(valid, z, 0.0))
        plsc.store_scatter(feat, [row, cols[3]], jnp.where(valid, w, 0.0))
        plsc.store_scatter(feat, [row, cols[4]],
                           jnp.where(valid, jnp.float32(1.0), jnp.float32(0.0)))
        jrow = jnp.full((L,), i // 8, jnp.int32)
        jcol = jnp.full((L,), (i % 8) * L, jnp.int32) + iota
        plsc.store_scatter(idx, [jrow, jcol], bin_)
        return carry

      lax.fori_loop(0, CH // L, step, 0)
      for j in range(NSTREAM):
        pltpu.sync_copy(feat.at[pl.ds(j * 128, 128)], table.at[idx.at[j]],
                        add=True)
    plsc.subcore_barrier()
    pltpu.sync_copy(table.at[pl.ds(z0, RZ)],
                    partials.at[c, pl.ds(z0, RZ)])

  return body(chans, zeros)


def _combine_kernel(partials):
  @pl.kernel(
      out_type=jax.ShapeDtypeStruct((NB * 4,), jnp.float32),
      mesh=_mesh,
      compiler_params=pltpu.CompilerParams(needs_layout_passes=False, use_tc_tiling_on_sc=False),
      scratch_types=[
          pltpu.VMEM((RB, 8), jnp.float32),
          pltpu.VMEM((RB, 8), jnp.float32),
          pltpu.VMEM((RB * 4,), jnp.float32),
      ],
  )
  def body(partials_hbm, out, bufa, bufb, obuf):
    c = lax.axis_index("c")
    s = lax.axis_index("s")
    wid = s * NC + c
    r0 = pl.multiple_of(wid * RB, RB)
    pltpu.sync_copy(partials_hbm.at[0, pl.ds(r0, RB)], bufa)
    pltpu.sync_copy(partials_hbm.at[1, pl.ds(r0, RB)], bufb)
    iota = lax.iota(jnp.int32, L)
    rbase = iota // 4
    fcol = iota % 4
    c4 = jnp.full((L,), 4, jnp.int32)

    def step(i, carry):
      rv = jnp.full((L,), i * 4, jnp.int32) + rbase
      fa = plsc.load_gather(bufa, [rv, fcol])
      fb = plsc.load_gather(bufb, [rv, fcol])
      ca = plsc.load_gather(bufa, [rv, c4])
      cb = plsc.load_gather(bufb, [rv, c4])
      cnt = jnp.maximum(ca + cb, jnp.float32(1.0))
      obuf[pl.ds(i * L, L)] = (fa + fb) / cnt
      return carry

    lax.fori_loop(0, RB * 4 // L, step, 0)
    o0 = pl.multiple_of(wid * (RB * 4), RB * 4)
    pltpu.sync_copy(obuf, out.at[pl.ds(o0, RB * 4)])

  return body(partials)


def kernel(clouds):
  B, C, N = clouds.shape
  chans = jnp.pad(clouds, ((0, 0), (0, 0), (0, NPAD - N)),
                  constant_values=-1e9).reshape(B * C, NPAD)
  zeros = jnp.zeros((RZ, 8), jnp.float32)
  partials = _scatter_kernel(chans, zeros)
  return _combine_kernel(partials).reshape(NB, 4)
